# baseline (speedup 1.0000x reference)
import functools

import jax
import jax.numpy as jnp
from jax import lax
from jax.experimental import pallas as pl
from jax.experimental.pallas import tpu as pltpu
from jax.experimental.pallas import tpu_sc as plsc

B, S, D = 4, 8192, 768
NC, NS, L = 2, 16, 16
NW = NC * NS
N_ROWS = B * S
ROWS_PER_W = N_ROWS // NW
TILE_ROWS = 32
TILE_F = TILE_ROWS * D
N_STEPS = ROWS_PER_W // TILE_ROWS


def _body(x_hbm, t_hbm, o_hbm, xv, tv):
    c = lax.axis_index("c")
    s = lax.axis_index("s")
    wid = s * NC + c
    row0 = wid * ROWS_PER_W
    srow0 = lax.rem(row0, S)

    def step(j, _):
        base = (row0 + j * TILE_ROWS) * D
        tbase = (srow0 + j * TILE_ROWS) * D
        pltpu.sync_copy(x_hbm.at[pl.ds(base, TILE_F)], xv)
        pltpu.sync_copy(t_hbm.at[pl.ds(tbase, TILE_F)], tv)

        @plsc.parallel_loop(0, TILE_F, L, unroll=8)
        def _add(v):
            sl = pl.ds(v, L)
            xv[sl] = xv[sl] + tv[sl]
        pltpu.sync_copy(xv, o_hbm.at[pl.ds(base, TILE_F)])
        return ()

    lax.fori_loop(0, N_STEPS, step, ())


@jax.jit
def kernel(inputs, pos_table):
    x = inputs.reshape(N_ROWS * D)
    t = pos_table.reshape(S * D)
    mesh = plsc.VectorSubcoreMesh(core_axis_name="c", subcore_axis_name="s",
                                  num_cores=NC, num_subcores=NS)
    out = pl.kernel(
        _body,
        out_type=jax.ShapeDtypeStruct((N_ROWS * D,), jnp.float32),
        mesh=mesh,
        scratch_types=[
            pltpu.VMEM((TILE_F,), jnp.float32),
            pltpu.VMEM((TILE_F,), jnp.float32),
        ],
    )(x, t)
    return out.reshape(B, S, D)


# speedup vs baseline: 1.3139x; 1.3139x over previous
import jax
import jax.numpy as jnp
from jax import lax
from jax.experimental import pallas as pl
from jax.experimental.pallas import tpu as pltpu
from jax.experimental.pallas import tpu_sc as plsc

B, S, D = 4, 8192, 768
NC, NS, L = 2, 16, 16
NW = NC * NS
SROWS_PER_W = S // NW
TILE_ROWS = 32
TILE_F = TILE_ROWS * D
NT = SROWS_PER_W // TILE_ROWS
NSTEP = NT * B


def _body(x_hbm, t_hbm, o_hbm, ta, tb, xa, xb,
          sem_ta, sem_tb, sem_la, sem_lb, sem_sa, sem_sb):
    c = lax.axis_index("c")
    s = lax.axis_index("s")
    wid = s * NC + c
    srow0 = wid * SROWS_PER_W

    tbufs, tsems = (ta, tb), (sem_ta, sem_tb)
    xbufs, lsems, ssems = (xa, xb), (sem_la, sem_lb), (sem_sa, sem_sb)

    def t_off(tile):
        return (srow0 + tile * TILE_ROWS) * D

    def x_off(k):
        tile, b = k // B, k % B
        return b * S * D + (srow0 + tile * TILE_ROWS) * D

    tload = {0: pltpu.async_copy(t_hbm.at[pl.ds(t_off(0), TILE_F)],
                                 tbufs[0], tsems[0])}
    xload = {0: pltpu.async_copy(x_hbm.at[pl.ds(x_off(0), TILE_F)],
                                 xbufs[0], lsems[0])}
    store = {}

    for k in range(NSTEP):
        xi = k % 2
        if k + 1 < NSTEP:
            if k - 1 >= 0:
                store.pop(k - 1).wait()
            xload[k + 1] = pltpu.async_copy(
                x_hbm.at[pl.ds(x_off(k + 1), TILE_F)],
                xbufs[(k + 1) % 2], lsems[(k + 1) % 2])
            if (k + 1) % B == 0:
                nt_ = (k + 1) // B
                tload[nt_] = pltpu.async_copy(
                    t_hbm.at[pl.ds(t_off(nt_), TILE_F)],
                    tbufs[nt_ % 2], tsems[nt_ % 2])
        tile = k // B
        if k % B == 0:
            tload.pop(tile).wait()
        xload.pop(k).wait()

        tv, xv = tbufs[tile % 2], xbufs[xi]

        @plsc.parallel_loop(0, TILE_F, L, unroll=8)
        def _add(v):
            sl = pl.ds(v, L)
            xv[sl] = xv[sl] + tv[sl]

        store[k] = pltpu.async_copy(xv, o_hbm.at[pl.ds(x_off(k), TILE_F)],
                                    ssems[xi])

    store.pop(NSTEP - 2).wait()
    store.pop(NSTEP - 1).wait()


@jax.jit
def kernel(inputs, pos_table):
    x = inputs.reshape(B * S * D)
    t = pos_table.reshape(S * D)
    mesh = plsc.VectorSubcoreMesh(core_axis_name="c", subcore_axis_name="s",
                                  num_cores=NC, num_subcores=NS)
    out = pl.kernel(
        _body,
        out_type=jax.ShapeDtypeStruct((B * S * D,), jnp.float32),
        mesh=mesh,
        scratch_types=[
            pltpu.VMEM((TILE_F,), jnp.float32),
            pltpu.VMEM((TILE_F,), jnp.float32),
            pltpu.VMEM((TILE_F,), jnp.float32),
            pltpu.VMEM((TILE_F,), jnp.float32),
            pltpu.SemaphoreType.DMA,
            pltpu.SemaphoreType.DMA,
            pltpu.SemaphoreType.DMA,
            pltpu.SemaphoreType.DMA,
            pltpu.SemaphoreType.DMA,
            pltpu.SemaphoreType.DMA,
        ],
    )(x, t)
    return out.reshape(B, S, D)


# speedup vs baseline: 3.6135x; 2.7502x over previous
import jax
import jax.numpy as jnp
from jax import lax
from jax.experimental import pallas as pl
from jax.experimental.pallas import tpu as pltpu
from jax.experimental.pallas import tpu_sc as plsc

B, S, D = 4, 8192, 768
NC, NS, L = 2, 16, 16
NW = NC * NS
SROWS_PER_W = S // NW
TILE_ROWS = 32
TILE_F = TILE_ROWS * D
NT = SROWS_PER_W // TILE_ROWS
NSTEP = NT * B


def _body(x_hbm, t_hbm, o_hbm, ta, tb, xa, xb,
          sem_ta, sem_tb, sem_la, sem_lb, sem_sa, sem_sb):
    c = lax.axis_index("c")
    s = lax.axis_index("s")
    wid = s * NC + c
    srow0 = wid * SROWS_PER_W

    tbufs, tsems = (ta, tb), (sem_ta, sem_tb)
    xbufs, lsems, ssems = (xa, xb), (sem_la, sem_lb), (sem_sa, sem_sb)

    def t_slice(tile):
        return t_hbm.at[pl.ds(srow0 + tile * TILE_ROWS, TILE_ROWS), :]

    def x_slice(ref, k):
        tile, b = k // B, k % B
        return ref.at[b, pl.ds(srow0 + tile * TILE_ROWS, TILE_ROWS), :]

    tload = {0: pltpu.async_copy(t_slice(0), tbufs[0], tsems[0])}
    xload = {0: pltpu.async_copy(x_slice(x_hbm, 0), xbufs[0], lsems[0])}
    store = {}

    for k in range(NSTEP):
        xi = k % 2
        if k + 1 < NSTEP:
            if k - 1 >= 0:
                store.pop(k - 1).wait()
            xload[k + 1] = pltpu.async_copy(
                x_slice(x_hbm, k + 1), xbufs[(k + 1) % 2],
                lsems[(k + 1) % 2])
            if (k + 1) % B == 0:
                nt_ = (k + 1) // B
                tload[nt_] = pltpu.async_copy(
                    t_slice(nt_), tbufs[nt_ % 2], tsems[nt_ % 2])
        tile = k // B
        if k % B == 0:
            tload.pop(tile).wait()
        xload.pop(k).wait()

        tv, xv = tbufs[tile % 2], xbufs[xi]

        @plsc.parallel_loop(0, TILE_ROWS, 1)
        def _add(r):
            @plsc.parallel_loop(0, D, L, unroll=8)
            def _add_cols(c):
                sl = pl.ds(c, L)
                xv[r, sl] = xv[r, sl] + tv[r, sl]

        store[k] = pltpu.async_copy(xv, x_slice(o_hbm, k), ssems[xi])

    store.pop(NSTEP - 2).wait()
    store.pop(NSTEP - 1).wait()


@jax.jit
def kernel(inputs, pos_table):
    mesh = plsc.VectorSubcoreMesh(core_axis_name="c", subcore_axis_name="s",
                                  num_cores=NC, num_subcores=NS)
    return pl.kernel(
        _body,
        out_type=jax.ShapeDtypeStruct((B, S, D), jnp.float32),
        mesh=mesh,
        scratch_types=[
            pltpu.VMEM((TILE_ROWS, D), jnp.float32),
            pltpu.VMEM((TILE_ROWS, D), jnp.float32),
            pltpu.VMEM((TILE_ROWS, D), jnp.float32),
            pltpu.VMEM((TILE_ROWS, D), jnp.float32),
            pltpu.SemaphoreType.DMA,
            pltpu.SemaphoreType.DMA,
            pltpu.SemaphoreType.DMA,
            pltpu.SemaphoreType.DMA,
            pltpu.SemaphoreType.DMA,
            pltpu.SemaphoreType.DMA,
        ],
    )(inputs, pos_table)
